# initial kernel scaffold (unmeasured)
import jax
import jax.numpy as jnp
from jax import lax
from jax.experimental import pallas as pl
from jax.experimental.pallas import tpu as pltpu

N_DEV = 32


def kernel(x, w_mat):
    m_total, k_per = x.shape
    k_total, n_out = w_mat.shape
    m_per = m_total // N_DEV

    def body(x_ref, w_ref, out_ref,
             xbf_ref, xstage_ref, amax_ref,
             send_sems, recv_sems, amax_send_sems, amax_recv_sems):
        me = lax.axis_index("i")

        barrier_sem = pltpu.get_barrier_semaphore()
        for d in range(1, N_DEV):
            pl.semaphore_signal(
                barrier_sem, inc=1,
                device_id=((me + d) % N_DEV,),
                device_id_type=pl.DeviceIdType.MESH,
            )
        pl.semaphore_wait(barrier_sem, N_DEV - 1)

        xbf_ref[...] = x_ref[...].astype(jnp.bfloat16)

        for d in range(1, N_DEV):
            p = (me + d) % N_DEV
            rdma = pltpu.make_async_remote_copy(
                src_ref=xbf_ref.at[pl.ds(p * m_per, m_per), :],
                dst_ref=xstage_ref.at[me],
                send_sem=send_sems.at[p],
                recv_sem=recv_sems.at[me],
                device_id=(p,),
                device_id_type=pl.DeviceIdType.MESH,
            )
            rdma.start()

        xstage_ref[me] = xbf_ref[pl.ds(me * m_per, m_per), :]

        for d in range(1, N_DEV):
            q = (me - d) % N_DEV
            recv = pltpu.make_async_remote_copy(
                src_ref=xbf_ref.at[pl.ds(q * m_per, m_per), :],
                dst_ref=xstage_ref.at[q],
                send_sem=send_sems.at[q],
                recv_sem=recv_sems.at[q],
                device_id=(q,),
                device_id_type=pl.DeviceIdType.MESH,
            )
            recv.wait_recv()

        y = jnp.zeros((m_per, n_out), jnp.float32)
        for q in range(N_DEV):
            w_blk = w_ref[pl.ds(q * k_per, k_per), :].astype(jnp.bfloat16)
            y = y + lax.dot_general(
                xstage_ref[q], w_blk,
                dimension_numbers=(((1,), (0,)), ((), ())),
                preferred_element_type=jnp.float32,
            )

        local_amax = jnp.max(jnp.abs(y))
        amax_ref[me] = jnp.full((8, 128), local_amax, jnp.float32)
        for d in range(1, N_DEV):
            p = (me + d) % N_DEV
            rdma = pltpu.make_async_remote_copy(
                src_ref=amax_ref.at[me],
                dst_ref=amax_ref.at[me],
                send_sem=amax_send_sems.at[p],
                recv_sem=amax_recv_sems.at[me],
                device_id=(p,),
                device_id_type=pl.DeviceIdType.MESH,
            )
            rdma.start()
        for d in range(1, N_DEV):
            q = (me - d) % N_DEV
            recv = pltpu.make_async_remote_copy(
                src_ref=amax_ref.at[q],
                dst_ref=amax_ref.at[q],
                send_sem=amax_send_sems.at[q],
                recv_sem=amax_recv_sems.at[q],
                device_id=(q,),
                device_id_type=pl.DeviceIdType.MESH,
            )
            recv.wait_recv()
        global_amax = jnp.max(amax_ref[...])

        scale = global_amax / 127.0
        qv = jnp.clip(jnp.round(y / scale), -127.0, 127.0)
        out_ref[...] = qv * scale

        for d in range(1, N_DEV):
            p = (me + d) % N_DEV
            s = pltpu.make_async_remote_copy(
                src_ref=xbf_ref.at[pl.ds(p * m_per, m_per), :],
                dst_ref=xstage_ref.at[me],
                send_sem=send_sems.at[p],
                recv_sem=recv_sems.at[me],
                device_id=(p,),
                device_id_type=pl.DeviceIdType.MESH,
            )
            s.wait_send()
            s2 = pltpu.make_async_remote_copy(
                src_ref=amax_ref.at[me],
                dst_ref=amax_ref.at[me],
                send_sem=amax_send_sems.at[p],
                recv_sem=amax_recv_sems.at[me],
                device_id=(p,),
                device_id_type=pl.DeviceIdType.MESH,
            )
            s2.wait_send()

    return pl.pallas_call(
        body,
        out_shape=jax.ShapeDtypeStruct((m_per, n_out), jnp.float32),
        in_specs=[
            pl.BlockSpec(memory_space=pltpu.VMEM),
            pl.BlockSpec(memory_space=pltpu.VMEM),
        ],
        out_specs=pl.BlockSpec(memory_space=pltpu.VMEM),
        scratch_shapes=[
            pltpu.VMEM((m_total, k_per), jnp.bfloat16),
            pltpu.VMEM((N_DEV, m_per, k_per), jnp.bfloat16),
            pltpu.VMEM((N_DEV, 8, 128), jnp.float32),
            pltpu.SemaphoreType.DMA((N_DEV,)),
            pltpu.SemaphoreType.DMA((N_DEV,)),
            pltpu.SemaphoreType.DMA((N_DEV,)),
            pltpu.SemaphoreType.DMA((N_DEV,)),
        ],
        compiler_params=pltpu.CompilerParams(collective_id=0),
    )(x, w_mat)


# baseline (device time: 43729 ns/iter reference)
import jax
import jax.numpy as jnp
from jax import lax
from jax.experimental import pallas as pl
from jax.experimental.pallas import tpu as pltpu

N_DEV = 32


def kernel(x, w_mat):
    m_total, k_per = x.shape
    k_total, n_out = w_mat.shape
    m_per = m_total // N_DEV

    def body(x_ref, w_ref, out_ref,
             xbf_ref, xstage_ref, amax_ref,
             send_sems, recv_sems, amax_send_sems, amax_recv_sems):
        me = lax.axis_index("i")

        barrier_sem = pltpu.get_barrier_semaphore()
        for d in range(1, N_DEV):
            pl.semaphore_signal(
                barrier_sem, inc=1,
                device_id=((me + d) % N_DEV,),
                device_id_type=pl.DeviceIdType.MESH,
            )
        pl.semaphore_wait(barrier_sem, N_DEV - 1)

        xbf_ref[...] = x_ref[...].astype(jnp.bfloat16)

        for d in range(1, N_DEV):
            p = (me + d) % N_DEV
            rdma = pltpu.make_async_remote_copy(
                src_ref=xbf_ref.at[pl.ds(p * m_per, m_per), :],
                dst_ref=xstage_ref.at[me],
                send_sem=send_sems.at[p],
                recv_sem=recv_sems.at[me],
                device_id=(p,),
                device_id_type=pl.DeviceIdType.MESH,
            )
            rdma.start()

        xstage_ref[me] = xbf_ref[pl.ds(me * m_per, m_per), :]

        for d in range(1, N_DEV):
            q = (me - d) % N_DEV
            recv = pltpu.make_async_remote_copy(
                src_ref=xbf_ref.at[pl.ds(q * m_per, m_per), :],
                dst_ref=xstage_ref.at[q],
                send_sem=send_sems.at[q],
                recv_sem=recv_sems.at[q],
                device_id=(q,),
                device_id_type=pl.DeviceIdType.MESH,
            )
            recv.wait_recv()

        y = jnp.zeros((m_per, n_out), jnp.float32)
        for q in range(N_DEV):
            w_blk = w_ref[pl.ds(q * k_per, k_per), :].astype(jnp.bfloat16)
            y = y + lax.dot_general(
                xstage_ref[q], w_blk,
                dimension_numbers=(((1,), (0,)), ((), ())),
                preferred_element_type=jnp.float32,
            )

        local_amax = jnp.max(jnp.abs(y))
        amax_ref[me] = jnp.full((8, 128), local_amax, jnp.float32)
        for d in range(1, N_DEV):
            p = (me + d) % N_DEV
            rdma = pltpu.make_async_remote_copy(
                src_ref=amax_ref.at[me],
                dst_ref=amax_ref.at[me],
                send_sem=amax_send_sems.at[p],
                recv_sem=amax_recv_sems.at[me],
                device_id=(p,),
                device_id_type=pl.DeviceIdType.MESH,
            )
            rdma.start()
        for d in range(1, N_DEV):
            q = (me - d) % N_DEV
            recv = pltpu.make_async_remote_copy(
                src_ref=amax_ref.at[q],
                dst_ref=amax_ref.at[q],
                send_sem=amax_send_sems.at[q],
                recv_sem=amax_recv_sems.at[q],
                device_id=(q,),
                device_id_type=pl.DeviceIdType.MESH,
            )
            recv.wait_recv()
        global_amax = jnp.max(amax_ref[...])

        scale = global_amax / 127.0
        qv = jnp.clip(jnp.round(y / scale), -127.0, 127.0)
        out_ref[...] = qv * scale

        for d in range(1, N_DEV):
            p = (me + d) % N_DEV
            s = pltpu.make_async_remote_copy(
                src_ref=xbf_ref.at[pl.ds(p * m_per, m_per), :],
                dst_ref=xstage_ref.at[me],
                send_sem=send_sems.at[p],
                recv_sem=recv_sems.at[me],
                device_id=(p,),
                device_id_type=pl.DeviceIdType.MESH,
            )
            s.wait_send()
            s2 = pltpu.make_async_remote_copy(
                src_ref=amax_ref.at[me],
                dst_ref=amax_ref.at[me],
                send_sem=amax_send_sems.at[p],
                recv_sem=amax_recv_sems.at[me],
                device_id=(p,),
                device_id_type=pl.DeviceIdType.MESH,
            )
            s2.wait_send()

    return pl.pallas_call(
        body,
        out_shape=jax.ShapeDtypeStruct((m_per, n_out), jnp.float32),
        in_specs=[
            pl.BlockSpec(memory_space=pltpu.VMEM),
            pl.BlockSpec(memory_space=pltpu.VMEM),
        ],
        out_specs=pl.BlockSpec(memory_space=pltpu.VMEM),
        scratch_shapes=[
            pltpu.VMEM((m_total, k_per), jnp.bfloat16),
            pltpu.VMEM((N_DEV, m_per, k_per), jnp.bfloat16),
            pltpu.VMEM((N_DEV, 8, 128), jnp.float32),
            pltpu.SemaphoreType.DMA((N_DEV,)),
            pltpu.SemaphoreType.DMA((N_DEV,)),
            pltpu.SemaphoreType.DMA((N_DEV,)),
            pltpu.SemaphoreType.DMA((N_DEV,)),
        ],
        compiler_params=pltpu.CompilerParams(
            collective_id=0,
            vmem_limit_bytes=60 * 1024 * 1024,
        ),
    )(x, w_mat)
